# baseline (speedup 1.0000x reference)
import functools

import jax
import jax.numpy as jnp
from jax import lax
from jax.experimental import pallas as pl
from jax.experimental.pallas import tpu as pltpu
from jax.experimental.pallas import tpu_sc as plsc

N = 8388608
NBINS = 65536
NC = 2
NS = 16
NW = NC * NS
NPW = N // NW
NPW2 = 2 * NPW
CHUNK = 16384
NCHUNKS = NPW // CHUNK
UNROLL = 8

def _i32(v):
    return jnp.int32(v)


def _make_hist_body(npw):
    nchunks = npw // CHUNK

    def _hist_body(x_hbm, out_hbm, hist, buf0, buf1, sem0, sem1):
        cid = lax.axis_index("c").astype(jnp.int32)
        sid = lax.axis_index("s").astype(jnp.int32)
        wid = sid * _i32(NC) + cid
        base = wid * _i32(npw)

        zeros = jnp.zeros((16,), jnp.int32)

        def zero_body(i):
            hist[pl.ds(i * _i32(16), 16)] = zeros

        plsc.parallel_loop(_i32(0), _i32(NBINS // 16), _i32(1), unroll=8)(zero_body)

        ones = jnp.ones((16,), jnp.int32)
        sems = [sem0, sem1]
        bufs = [buf0, buf1]

        copies = [None, None]
        copies[0] = pltpu.async_copy(
            x_hbm.at[pl.ds(base, CHUNK)], bufs[0], sems[0])
        for k in range(nchunks):
            cur = k % 2
            nxt = (k + 1) % 2
            if k + 1 < nchunks:
                copies[nxt] = pltpu.async_copy(
                    x_hbm.at[pl.ds(base + _i32((k + 1) * CHUNK), CHUNK)],
                    bufs[nxt], sems[nxt])
            copies[cur].wait()
            b = bufs[cur]

            def chunk_body(j):
                idx = plsc.bitcast(b[pl.ds(j * _i32(16), 16)], jnp.int32)
                plsc.addupdate_scatter(hist, [idx], ones)

            plsc.parallel_loop(_i32(0), _i32(CHUNK // 16), _i32(1),
                               unroll=UNROLL)(chunk_body)

        pltpu.sync_copy(hist, out_hbm.at[wid])

    return _hist_body


@functools.cache
def _sc_hist(npw):
    mesh = plsc.VectorSubcoreMesh(
        core_axis_name="c", subcore_axis_name="s", num_cores=NC, num_subcores=NS)
    return pl.kernel(
        _make_hist_body(npw),
        out_type=jax.ShapeDtypeStruct((NW, NBINS), jnp.int32),
        mesh=mesh,
        scratch_types=[
            pltpu.VMEM((NBINS,), jnp.int32),
            pltpu.VMEM((CHUNK,), jnp.uint32),
            pltpu.VMEM((CHUNK,), jnp.uint32),
            pltpu.SemaphoreType.DMA,
            pltpu.SemaphoreType.DMA,
        ],
        compiler_params=pltpu.CompilerParams(needs_layout_passes=False),
    )


NPHASES = 4


def _merge_body(*refs):
    o_ref = refs[-1]
    acc = jnp.sum(refs[0][...], axis=0, dtype=jnp.int32)
    for r in refs[1:-1]:
        acc = acc + jnp.sum(r[...], axis=0, dtype=jnp.int32)
    o_ref[...] = acc


def kernel(x):
    np_phase = N // NPHASES
    partials = []
    for i in range(NPHASES):
        xs = lax.slice(x, (i * np_phase,), ((i + 1) * np_phase,))
        partials.append(_sc_hist(np_phase // NW)(xs.astype(jnp.uint32)))
    ps = [p.reshape(NW, NBINS // 128, 128) for p in partials]
    merged = pl.pallas_call(
        _merge_body,
        out_shape=jax.ShapeDtypeStruct((NBINS // 128, 128), jnp.int32),
    )(*ps)
    return merged.reshape(NBINS).astype(jnp.int64)


# speedup vs baseline: 1.3327x; 1.3327x over previous
import functools

import jax
import jax.numpy as jnp
from jax import lax
from jax.experimental import pallas as pl
from jax.experimental.pallas import tpu as pltpu
from jax.experimental.pallas import tpu_sc as plsc

N = 8388608
NBINS = 65536
NC = 2
NS = 16
NW = NC * NS
NPW = N // NW
NPW2 = 2 * NPW
CHUNK = 16384
NCHUNKS = NPW // CHUNK
UNROLL = 8

def _i32(v):
    return jnp.int32(v)


def _hist_body(x_hbm, out_hbm, hist, buf0, buf1, sem0, sem1):
    cid = lax.axis_index("c").astype(jnp.int32)
    sid = lax.axis_index("s").astype(jnp.int32)
    wid = sid * _i32(NC) + cid
    base = wid * _i32(NPW)

    zeros = jnp.zeros((16,), jnp.int32)

    def zero_body(i):
        hist[pl.ds(i * _i32(16), 16)] = zeros

    plsc.parallel_loop(_i32(0), _i32(NBINS // 16), _i32(1), unroll=8)(zero_body)

    ones = jnp.ones((16,), jnp.int32)
    sems = [sem0, sem1]
    bufs = [buf0, buf1]

    copies = [None, None]
    copies[0] = pltpu.async_copy(
        x_hbm.at[pl.ds(base, CHUNK)], bufs[0], sems[0])
    for k in range(NCHUNKS):
        cur = k % 2
        nxt = (k + 1) % 2
        if k + 1 < NCHUNKS:
            copies[nxt] = pltpu.async_copy(
                x_hbm.at[pl.ds(base + _i32((k + 1) * CHUNK), CHUNK)],
                bufs[nxt], sems[nxt])
        copies[cur].wait()
        b = bufs[cur]

        def chunk_body(j):
            idx = plsc.bitcast(b[pl.ds(j * _i32(16), 16)], jnp.int32)
            plsc.addupdate_scatter(hist, [idx], ones)

        plsc.parallel_loop(_i32(0), _i32(CHUNK // 16), _i32(1), unroll=UNROLL)(chunk_body)

    pltpu.sync_copy(hist, out_hbm.at[pl.ds(wid * _i32(NBINS), NBINS)])


@functools.cache
def _sc_hist():
    mesh = plsc.VectorSubcoreMesh(
        core_axis_name="c", subcore_axis_name="s", num_cores=NC, num_subcores=NS)
    return pl.kernel(
        _hist_body,
        out_type=jax.ShapeDtypeStruct((NW * NBINS,), jnp.int32),
        mesh=mesh,
        scratch_types=[
            pltpu.VMEM((NBINS,), jnp.int32),
            pltpu.VMEM((CHUNK,), jnp.uint32),
            pltpu.VMEM((CHUNK,), jnp.uint32),
            pltpu.SemaphoreType.DMA,
            pltpu.SemaphoreType.DMA,
        ],
        compiler_params=pltpu.CompilerParams(needs_layout_passes=False),
    )


def _merge_body(p_ref, o_ref):
    o_ref[...] = jnp.sum(p_ref[...], axis=0, dtype=jnp.int32)


def kernel(x):
    xi = x.astype(jnp.uint32)
    partials = _sc_hist()(xi)
    p3 = partials.reshape(NW, NBINS // 128, 128)
    merged = pl.pallas_call(
        _merge_body,
        out_shape=jax.ShapeDtypeStruct((NBINS // 128, 128), jnp.int32),
    )(p3)
    return merged.reshape(NBINS).astype(jnp.int64)
